# baseline (speedup 1.0000x reference)
import functools

import jax
import jax.numpy as jnp
from jax import lax
from jax.experimental import pallas as pl
from jax.experimental.pallas import tpu as pltpu
from jax.experimental.pallas import tpu_sc as plsc


def _make_sc_add(B, L, D):
    NC, NS = 2, 16
    NW = NC * NS
    LD = L * D
    SLAB = LD // NW
    CHUNK = 16384
    NCHUNK = SLAB // CHUNK
    P = NCHUNK * B
    NXB = 4
    NEB = 2
    AHEAD = 2

    mesh = plsc.VectorSubcoreMesh(core_axis_name="c", subcore_axis_name="s")

    @functools.partial(
        pl.kernel,
        mesh=mesh,
        out_type=jax.ShapeDtypeStruct((B * LD,), jnp.float32),
        scratch_types=(
            [pltpu.VMEM((CHUNK,), jnp.float32) for _ in range(NXB)]
            + [pltpu.VMEM((CHUNK,), jnp.float32) for _ in range(NEB)]
            + [pltpu.SemaphoreType.DMA for _ in range(NXB + NXB + NEB)]
        ),
    )
    def sc_add(x_hbm, emb_hbm, out_hbm, *bufs):
        xb = bufs[:NXB]
        eb = bufs[NXB:NXB + NEB]
        sems = bufs[NXB + NEB:]
        in_sems = sems[:NXB]
        out_sems = sems[NXB:NXB + NXB]
        emb_sems = sems[NXB + NXB:]

        wid = lax.axis_index("s") * NC + lax.axis_index("c")
        base = wid * SLAB

        in_cp = [None] * P
        out_cp = [None] * P
        emb_cp = [None] * NCHUNK

        def x_off(p):
            c, b = divmod(p, B)
            return b * LD + base + c * CHUNK

        def start_in(p):
            in_cp[p] = pltpu.async_copy(
                x_hbm.at[pl.ds(x_off(p), CHUNK)], xb[p % NXB], in_sems[p % NXB]
            )
            c, b = divmod(p, B)
            if b == 0:
                emb_cp[c] = pltpu.async_copy(
                    emb_hbm.at[pl.ds(base + c * CHUNK, CHUNK)],
                    eb[c % NEB],
                    emb_sems[c % NEB],
                )

        for p in range(min(AHEAD, P)):
            start_in(p)

        for p in range(P):
            pn = p + AHEAD
            if pn < P:
                if pn >= NXB:
                    out_cp[pn - NXB].wait()
                start_in(pn)
            c, b = divmod(p, B)
            in_cp[p].wait()
            if b == 0:
                emb_cp[c].wait()
            xbuf = xb[p % NXB]
            ebuf = eb[c % NEB]

            @plsc.parallel_loop(0, CHUNK, 16, unroll=8)
            def _add(i):
                xbuf[pl.ds(i, 16)] = xbuf[pl.ds(i, 16)] + ebuf[pl.ds(i, 16)]

            out_cp[p] = pltpu.async_copy(
                xbuf, out_hbm.at[pl.ds(x_off(p), CHUNK)], out_sems[p % NXB]
            )

        for p in range(max(0, P - NXB), P):
            out_cp[p].wait()

    return sc_add


def kernel(x, emb_table):
    B, L, D = x.shape
    sc_add = _make_sc_add(B, L, D)
    out = sc_add(x.reshape(B * L * D), emb_table.reshape(L * D))
    return out.reshape(B, L, D)


# speedup vs baseline: 2.9695x; 2.9695x over previous
import functools

import jax
import jax.numpy as jnp
from jax import lax
from jax.experimental import pallas as pl
from jax.experimental.pallas import tpu as pltpu
from jax.experimental.pallas import tpu_sc as plsc


def _make_sc_add(B, L, D):
    NC, NS = 2, 16
    NW = NC * NS
    RW = L // NW
    CR = 16
    NCH = RW // CR
    NG = NCH // 2
    NV = D // 16
    NXB = 4
    P_GROUP = 2 * B

    mesh = plsc.VectorSubcoreMesh(core_axis_name="c", subcore_axis_name="s")

    @functools.partial(
        pl.kernel,
        mesh=mesh,
        out_type=jax.ShapeDtypeStruct((B, L, D), jnp.float32),
        scratch_types=(
            [pltpu.VMEM((CR, D), jnp.float32) for _ in range(NXB + 2)]
            + [pltpu.SemaphoreType.DMA for _ in range(NXB + NXB + 2)]
        ),
    )
    def sc_add(x_hbm, emb_hbm, out_hbm, *bufs):
        xb = bufs[:NXB]
        eb = bufs[NXB:NXB + 2]
        in_sems = bufs[NXB + 2:NXB + 2 + NXB]
        out_sems = bufs[NXB + 2 + NXB:NXB + 2 + 2 * NXB]
        emb_sems = bufs[NXB + 2 + 2 * NXB:]

        wid = lax.axis_index("s") * NC + lax.axis_index("c")
        row0 = wid * RW

        def in_start(rows, b, bufi):
            return pltpu.async_copy(
                x_hbm.at[b, pl.ds(rows, CR)], xb[bufi], in_sems[bufi]
            )

        def in_wait(bufi):
            pltpu.make_async_copy(
                x_hbm.at[0, pl.ds(0, CR)], xb[bufi], in_sems[bufi]
            ).wait()

        def out_start(rows, b, bufi):
            return pltpu.async_copy(
                xb[bufi], out_hbm.at[b, pl.ds(rows, CR)], out_sems[bufi]
            )

        def out_wait(bufi):
            pltpu.make_async_copy(
                xb[bufi], out_hbm.at[0, pl.ds(0, CR)], out_sems[bufi]
            ).wait()

        def emb_start(rows, ei):
            return pltpu.async_copy(
                emb_hbm.at[pl.ds(rows, CR)], eb[ei], emb_sems[ei]
            )

        def emb_wait(ei):
            pltpu.make_async_copy(
                emb_hbm.at[pl.ds(0, CR)], eb[ei], emb_sems[ei]
            ).wait()

        in_start(row0, 0, 0)
        in_start(row0, 1, 1)
        emb_start(row0, 0)
        emb_start(row0 + CR, 1)

        def group(gg, _):
            crow = [row0 + (2 * gg) * CR, row0 + (2 * gg + 1) * CR]
            nrow = [row0 + (2 * gg + 2) * CR, row0 + (2 * gg + 3) * CR]
            for j in range(P_GROUP):
                cj, b = divmod(j, B)
                bufi = j % NXB
                tgt = (j + 2) % NXB
                if j < 2:
                    @pl.when(gg != 0)
                    def _():
                        out_wait(tgt)
                    in_start(crow[(j + 2) // B], (j + 2) % B, tgt)
                elif j < P_GROUP - 2:
                    out_wait(tgt)
                    in_start(crow[(j + 2) // B], (j + 2) % B, tgt)
                else:
                    @pl.when(gg != NG - 1)
                    def _():
                        out_wait(tgt)
                        in_start(nrow[(j + 2 - P_GROUP) // B],
                                 (j + 2 - P_GROUP) % B, tgt)
                if j == 0:
                    emb_wait(0)
                if j == B:
                    emb_wait(1)
                in_wait(bufi)
                xbuf = xb[bufi]
                ebuf = eb[cj]

                @plsc.parallel_loop(0, CR, 1, unroll=2)
                def _add(r):
                    for k in range(NV):
                        sl = pl.ds(k * 16, 16)
                        xbuf[r, sl] = xbuf[r, sl] + ebuf[r, sl]

                out_start(crow[cj], b, bufi)
                if j == B - 1:
                    @pl.when(gg != NG - 1)
                    def _():
                        emb_start(nrow[0], 0)
                if j == P_GROUP - 1:
                    @pl.when(gg != NG - 1)
                    def _():
                        emb_start(nrow[1], 1)
            return 0

        lax.fori_loop(0, NG, group, 0)

        for i in range(NXB):
            out_wait(i)

    return sc_add


def kernel(x, emb_table):
    B, L, D = x.shape
    sc_add = _make_sc_add(B, L, D)
    return sc_add(x, emb_table)


# speedup vs baseline: 3.0372x; 1.0228x over previous
import functools

import jax
import jax.numpy as jnp
from jax import lax
from jax.experimental import pallas as pl
from jax.experimental.pallas import tpu as pltpu
from jax.experimental.pallas import tpu_sc as plsc


def _make_sc_add(B, L, D):
    NC, NS = 2, 16
    NW = NC * NS
    RW = L // NW
    CR = 8
    NCH = RW // CR
    NG = NCH // 2
    NV = D // 16

    mesh = plsc.VectorSubcoreMesh(core_axis_name="c", subcore_axis_name="s")

    @functools.partial(
        pl.kernel,
        mesh=mesh,
        out_type=jax.ShapeDtypeStruct((B, L, D), jnp.float32),
        scratch_types=(
            [pltpu.VMEM((CR, D), jnp.float32) for _ in range(2 * B)]
            + [pltpu.VMEM((CR, D), jnp.float32) for _ in range(2 * B)]
            + [pltpu.VMEM((CR, D), jnp.float32) for _ in range(2)]
            + [pltpu.SemaphoreType.DMA for _ in range(2 * B)]
            + [pltpu.SemaphoreType.DMA for _ in range(2 * B)]
            + [pltpu.SemaphoreType.DMA for _ in range(2)]
        ),
    )
    def sc_add(x_hbm, emb_hbm, out_hbm, *refs):
        xin = [refs[2 * b:2 * b + 2] for b in range(B)]
        xout = [refs[2 * B + 2 * b:2 * B + 2 * b + 2] for b in range(B)]
        eb = refs[4 * B:4 * B + 2]
        in_sems = [refs[4 * B + 2 + 2 * b:4 * B + 4 + 2 * b] for b in range(B)]
        out_sems = [refs[6 * B + 2 + 2 * b:6 * B + 4 + 2 * b] for b in range(B)]
        emb_sems = refs[8 * B + 2:]

        wid = lax.axis_index("s") * NC + lax.axis_index("c")
        row0 = wid * RW

        def ins_start(c, q):
            rows = pl.ds(row0 + c * CR, CR)
            pltpu.async_copy(emb_hbm.at[rows], eb[q], emb_sems[q])
            for b in range(B):
                pltpu.async_copy(x_hbm.at[b, rows], xin[b][q], in_sems[b][q])

        def ins_wait(q):
            pltpu.make_async_copy(
                emb_hbm.at[pl.ds(0, CR)], eb[q], emb_sems[q]
            ).wait()
            for b in range(B):
                pltpu.make_async_copy(
                    x_hbm.at[0, pl.ds(0, CR)], xin[b][q], in_sems[b][q]
                ).wait()

        def outs_start(c, q):
            rows = pl.ds(row0 + c * CR, CR)
            for b in range(B):
                pltpu.async_copy(xout[b][q], out_hbm.at[b, rows],
                                 out_sems[b][q])

        def outs_wait(q):
            for b in range(B):
                pltpu.make_async_copy(
                    xout[b][q], out_hbm.at[0, pl.ds(0, CR)], out_sems[b][q]
                ).wait()

        ins_start(0, 0)

        def group(gg, _):
            for j in range(2):
                c = 2 * gg + j
                q = j
                if j == 0:
                    ins_start(c + 1, 1)
                else:
                    @pl.when(gg != NG - 1)
                    def _():
                        ins_start(c + 1, 0)
                @pl.when(gg != 0)
                def _():
                    outs_wait(q)
                ins_wait(q)
                ebq = eb[q]
                xiq = [xin[b][q] for b in range(B)]
                xoq = [xout[b][q] for b in range(B)]

                @plsc.parallel_loop(0, CR, 1)
                def _add(r):
                    for k in range(NV):
                        sl = pl.ds(k * 16, 16)
                        ev = ebq[r, sl]
                        for b in range(B):
                            xoq[b][r, sl] = xiq[b][r, sl] + ev

                outs_start(c, q)
            return 0

        lax.fori_loop(0, NG, group, 0)

        outs_wait(0)
        outs_wait(1)

    return sc_add


def kernel(x, emb_table):
    B, L, D = x.shape
    sc_add = _make_sc_add(B, L, D)
    return sc_add(x, emb_table)
